# baseline (speedup 1.0000x reference)
import jax
import jax.numpy as jnp
from jax.experimental import pallas as pl
from jax.experimental.pallas import tpu as pltpu

_BM = 2000
_IN_BUFS = 4
_OUT_BUFS = 2


def kernel(adj, weight, bias):
    m, k = adj.shape
    n = weight.shape[1]
    bias2d = bias.reshape(1, n)

    def outer(adj_hbm, w_ref, b_ref, out_hbm):
        def body(a_ref, o_ref):
            o_ref[...] = (
                jnp.dot(a_ref[...], w_ref[...], preferred_element_type=jnp.float32)
                + b_ref[...]
            )

        pltpu.emit_pipeline(
            body,
            grid=(m // _BM,),
            in_specs=[
                pl.BlockSpec(
                    (_BM, k), lambda i: (i, 0),
                    pipeline_mode=pl.Buffered(buffer_count=_IN_BUFS),
                )
            ],
            out_specs=[
                pl.BlockSpec(
                    (_BM, n), lambda i: (i, 0),
                    pipeline_mode=pl.Buffered(buffer_count=_OUT_BUFS),
                )
            ],
        )(adj_hbm, out_hbm)

    return pl.pallas_call(
        outer,
        in_specs=[
            pl.BlockSpec(memory_space=pl.ANY),
            pl.BlockSpec((k, n), lambda: (0, 0)),
            pl.BlockSpec((1, n), lambda: (0, 0)),
        ],
        out_specs=pl.BlockSpec(memory_space=pl.ANY),
        out_shape=jax.ShapeDtypeStruct((m, n), jnp.float32),
    )(adj, weight, bias2d)
